# baseline (speedup 1.0000x reference)
import jax
import jax.numpy as jnp
from jax import lax
from jax.experimental import pallas as pl
from jax.experimental.pallas import tpu as pltpu
from jax.experimental.pallas import tpu_sc as plsc

VOCAB = 1000000
DIM = 64
BATCH = 16384
TOTAL = 327680

NC = 2
NS = 16
NW = NC * NS
BPW = BATCH // NW
PH = 4
BPP = BPW // PH
CH = 256
NSUB = CH // 128
OFF_PAD = 1032
IDX_PAD = 2 * CH + 8 + 504
GDUMMY = NS * BPP
RBLK = 2000

_params = pltpu.CompilerParams(
    needs_layout_passes=False, use_tc_tiling_on_sc=True)


def _stage_body(w_ref, o_ref):
    o_ref[:, :DIM] = w_ref[...]
    o_ref[:, DIM:] = jnp.zeros((RBLK, DIM), jnp.float32)


def _stage(weight):
    return pl.pallas_call(
        _stage_body,
        grid=(VOCAB // RBLK,),
        in_specs=[pl.BlockSpec((RBLK, DIM), lambda i: (i, 0))],
        out_specs=pl.BlockSpec((RBLK, 2 * DIM), lambda i: (i, 0)),
        out_shape=jax.ShapeDtypeStruct((VOCAB, 2 * DIM), jnp.float32),
    )(weight)


def _body(off_hbm, idx_hbm, w_hbm, out_hbm,
          off_v, idx_b, seg_b, bufs, zbuf, inv_v, acc_sh, gsems, ssems):
    sid = lax.axis_index("s")
    wid = sid * NC + lax.axis_index("c")
    bag0 = pl.multiple_of(wid * BPW, 8)
    abase = sid * BPP

    pltpu.sync_copy(off_hbm.at[pl.ds(bag0, OFF_PAD)], off_v)

    def _zero(r, _):
        for k in range(2 * DIM // 16):
            zbuf[r, pl.ds(k * 16, 16)] = jnp.zeros((16,), jnp.float32)
        return 0
    lax.fori_loop(0, BPP + 8, _zero, 0)

    lane = lax.iota(jnp.int32, 16)

    def _phase(h, _ph):
        sb = pl.multiple_of(h * BPP, 8)

        pltpu.sync_copy(zbuf.at[pl.ds(0, BPP)], acc_sh.at[pl.ds(abase, BPP)])

        @pl.when(sid == 0)
        def _():
            pltpu.sync_copy(zbuf.at[pl.ds(BPP, 8)],
                            acc_sh.at[pl.ds(GDUMMY, 8)])

        t0 = off_v[pl.ds(sb, 16)][0]
        t1 = off_v[pl.ds(sb + BPP, 16)][0]
        c0a = pl.multiple_of(lax.bitwise_and(t0, jnp.int32(-8)), 8)
        span = t1 - c0a
        nch = lax.div(span + (CH - 1), jnp.int32(CH))
        nchp = lax.div(nch + 1, jnp.int32(2))

        def _chunk(k, off):
            buf, gsem, ssem = bufs[off], gsems[off], ssems[off]
            idxp, segp = idx_b[off], seg_b[off]
            c0 = pl.multiple_of(c0a + k * CH, 8)

            @pl.when(k >= 2)
            def _():
                for j in range(NSUB):
                    pltpu.make_async_copy(
                        buf.at[pl.ds(128 * j, 128)],
                        acc_sh.at[segp[j]], ssem).wait()

            for j in range(NSUB):
                pltpu.sync_copy(idx_hbm.at[pl.ds(c0 + 128 * j, 128)],
                                idxp[j])
            descs = [pltpu.async_copy(w_hbm.at[idxp[j]],
                                      buf.at[pl.ds(128 * j, 128)], gsem)
                     for j in range(NSUB)]
            for j in range(NSUB):
                for q in range(128 // 16):
                    p = c0 + 128 * j + 16 * q + lane
                    c = jnp.zeros((16,), jnp.int32)
                    for s in (512, 256, 128, 64, 32, 16, 8, 4, 2, 1):
                        nc2 = c + s
                        val = plsc.load_gather(off_v, [nc2 - 1])
                        c = jnp.where(val <= p, nc2, c)
                    seg0 = c - 1
                    valid = (c > 0) & (seg0 >= sb) & (seg0 < sb + BPP)
                    aidx = jnp.where(valid, seg0 - (sb - abase), GDUMMY)
                    segp[j][pl.ds(16 * q, 16)] = aidx
            for d in descs:
                d.wait()
            for j in range(NSUB):
                pltpu.async_copy(buf.at[pl.ds(128 * j, 128)],
                                 acc_sh.at[segp[j]], ssem, add=True)

        def _pair(m, _):
            for off in range(2):
                _chunk(2 * m + off, off)
            return 0

        lax.fori_loop(0, nchp, _pair, 0)

        @pl.when(nchp > 0)
        def _():
            for off in range(2):
                for j in range(NSUB):
                    pltpu.make_async_copy(
                        bufs[off].at[pl.ds(128 * j, 128)],
                        acc_sh.at[seg_b[off][j]], ssems[off]).wait()

        for g in range(BPP // 16):
            a = plsc.load_gather(off_v, [lane + sb + g * 16])
            b = plsc.load_gather(off_v, [lane + sb + g * 16 + 1])
            cnt = (b - a).astype(jnp.float32)
            inv_v[pl.ds(g * 16, 16)] = 1.0 / jnp.maximum(cnt, 1.0)

        pltpu.sync_copy(acc_sh.at[pl.ds(abase, BPP)],
                        bufs[0].at[pl.ds(0, BPP)])

        def _scale(r, _):
            s = inv_v[pl.ds(r, 16)][0]
            for k in range(DIM // 16):
                bufs[0][r, pl.ds(k * 16, 16)] = (
                    bufs[0][r, pl.ds(k * 16, 16)] * s)
            return 0
        lax.fori_loop(0, BPP, _scale, 0)

        pltpu.sync_copy(bufs[0].at[pl.ds(0, BPP)],
                        out_hbm.at[pl.ds(bag0 + sb, BPP)])
        return 0

    lax.fori_loop(0, PH, _phase, 0)


@jax.jit
def _run(offsets_ext, indices_pad, staged):
    mesh = plsc.VectorSubcoreMesh(core_axis_name="c", subcore_axis_name="s")
    scratch = (
        pltpu.VMEM((OFF_PAD,), jnp.int32),
        [[pltpu.VMEM((128,), jnp.int32) for _ in range(NSUB)]
         for _ in range(2)],
        [[pltpu.VMEM((128,), jnp.int32) for _ in range(NSUB)]
         for _ in range(2)],
        [pltpu.VMEM((CH, 2 * DIM), jnp.float32)
         for _ in range(2)],
        pltpu.VMEM((BPP + 8, 2 * DIM), jnp.float32),
        pltpu.VMEM((BPP + 16,), jnp.float32),
        pltpu.VMEM_SHARED((NS * BPP + 8, 2 * DIM), jnp.float32),
        [pltpu.SemaphoreType.DMA for _ in range(2)],
        [pltpu.SemaphoreType.DMA for _ in range(2)],
    )
    return pl.kernel(
        _body,
        out_type=jax.ShapeDtypeStruct((BATCH, 2 * DIM), jnp.float32),
        mesh=mesh,
        scratch_types=scratch,
        compiler_params=_params,
    )(offsets_ext, indices_pad, staged)


def kernel(indices, offsets, weight):
    offsets_ext = jnp.concatenate(
        [offsets, jnp.full((OFF_PAD,), TOTAL, jnp.int32)])
    indices_pad = jnp.concatenate(
        [indices, jnp.zeros((IDX_PAD,), jnp.int32)])
    staged = _stage(weight)
    return _run(offsets_ext, indices_pad, staged)[:, :DIM]


# speedup vs baseline: 1.4208x; 1.4208x over previous
import jax
import jax.numpy as jnp
from jax import lax
from jax.experimental import pallas as pl
from jax.experimental.pallas import tpu as pltpu
from jax.experimental.pallas import tpu_sc as plsc

VOCAB = 1000000
DIM = 64
BATCH = 16384
TOTAL = 327680

NC = 2
NS = 16
NW = NC * NS
BPW = BATCH // NW
PH = 4
BPP = BPW // PH
CH = 256
NSUB = CH // 128
OFF_PAD = 1032
IDX_PAD = 2 * CH + 8 + 504
GDUMMY = NS * BPP
RBLK = 2000

_params = pltpu.CompilerParams(
    needs_layout_passes=False, use_tc_tiling_on_sc=True)


def _stage_body(w_ref, o_ref):
    o_ref[:, :DIM] = w_ref[...]
    o_ref[:, DIM:] = jnp.zeros((RBLK, DIM), jnp.float32)


def _stage(weight):
    return pl.pallas_call(
        _stage_body,
        grid=(VOCAB // RBLK,),
        in_specs=[pl.BlockSpec((RBLK, DIM), lambda i: (i, 0))],
        out_specs=pl.BlockSpec((RBLK, 2 * DIM), lambda i: (i, 0)),
        out_shape=jax.ShapeDtypeStruct((VOCAB, 2 * DIM), jnp.float32),
    )(weight)


def _body(off_hbm, idx_hbm, w_hbm, out_hbm,
          off_v, idx_b, seg_b, bufs, zbuf, inv_v, acc_sh, gsems, ssems):
    sid = lax.axis_index("s")
    wid = sid * NC + lax.axis_index("c")
    bag0 = pl.multiple_of(wid * BPW, 8)
    abase = sid * BPP

    pltpu.sync_copy(off_hbm.at[pl.ds(bag0, OFF_PAD)], off_v)

    def _zero(r, _):
        for k in range(2 * DIM // 16):
            zbuf[r, pl.ds(k * 16, 16)] = jnp.zeros((16,), jnp.float32)
        return 0
    lax.fori_loop(0, BPP + 8, _zero, 0)

    lane = lax.iota(jnp.int32, 16)

    def _phase(h, _ph):
        sb = pl.multiple_of(h * BPP, 8)

        pltpu.sync_copy(zbuf.at[pl.ds(0, BPP)], acc_sh.at[pl.ds(abase, BPP)])

        @pl.when(sid == 0)
        def _():
            pltpu.sync_copy(zbuf.at[pl.ds(BPP, 8)],
                            acc_sh.at[pl.ds(GDUMMY, 8)])

        t0 = off_v[pl.ds(sb, 16)][0]
        t1 = off_v[pl.ds(sb + BPP, 16)][0]
        c0a = pl.multiple_of(lax.bitwise_and(t0, jnp.int32(-8)), 8)
        span = t1 - c0a
        nch = lax.div(span + (CH - 1), jnp.int32(CH))
        nchp = lax.div(nch + 1, jnp.int32(2))

        def _chunk(k, off):
            buf, gsem, ssem = bufs[off], gsems[off], ssems[off]
            idxp, segp = idx_b[off], seg_b[off]
            c0 = pl.multiple_of(c0a + k * CH, 8)

            @pl.when(k >= 2)
            def _():
                for j in range(NSUB):
                    pltpu.make_async_copy(
                        buf.at[pl.ds(128 * j, 128)],
                        acc_sh.at[segp[j]], ssem).wait()

            for j in range(NSUB):
                pltpu.sync_copy(idx_hbm.at[pl.ds(c0 + 128 * j, 128)],
                                idxp[j])
            descs = [pltpu.async_copy(w_hbm.at[idxp[j]],
                                      buf.at[pl.ds(128 * j, 128)], gsem)
                     for j in range(NSUB)]
            for j in range(NSUB):
                for q in range(128 // 16):
                    p = c0 + 128 * j + 16 * q + lane
                    c = jnp.zeros((16,), jnp.int32)
                    for s in (512, 256, 128, 64, 32, 16, 8, 4, 2, 1):
                        nc2 = c + s
                        val = plsc.load_gather(off_v, [nc2 - 1])
                        c = jnp.where(val <= p, nc2, c)
                    seg0 = c - 1
                    valid = (c > 0) & (seg0 >= sb) & (seg0 < sb + BPP)
                    aidx = jnp.where(valid, seg0 - (sb - abase), GDUMMY)
                    segp[j][pl.ds(16 * q, 16)] = aidx
            for d in descs:
                d.wait()
            for j in range(NSUB):
                pltpu.async_copy(buf.at[pl.ds(128 * j, 128)],
                                 acc_sh.at[segp[j]], ssem, add=True)

        def _pair(m, _):
            for off in range(2):
                _chunk(2 * m + off, off)
            return 0

        lax.fori_loop(0, nchp, _pair, 0)

        @pl.when(nchp > 0)
        def _():
            for off in range(2):
                for j in range(NSUB):
                    pltpu.make_async_copy(
                        bufs[off].at[pl.ds(128 * j, 128)],
                        acc_sh.at[seg_b[off][j]], ssems[off]).wait()

        for g in range(BPP // 16):
            a = plsc.load_gather(off_v, [lane + sb + g * 16])
            b = plsc.load_gather(off_v, [lane + sb + g * 16 + 1])
            cnt = (b - a).astype(jnp.float32)
            inv_v[pl.ds(g * 16, 16)] = 1.0 / jnp.maximum(cnt, 1.0)

        pltpu.sync_copy(acc_sh.at[pl.ds(abase, BPP)],
                        bufs[0].at[pl.ds(0, BPP)])

        def _scale(r, _):
            s = inv_v[pl.ds(r, 16)][0]
            for k in range(DIM // 16):
                bufs[0][r, pl.ds(k * 16, 16)] = (
                    bufs[0][r, pl.ds(k * 16, 16)] * s)
            return 0
        lax.fori_loop(0, BPP, _scale, 0)

        pltpu.sync_copy(bufs[0].at[pl.ds(0, BPP)],
                        out_hbm.at[pl.ds(bag0 + sb, BPP)])
        return 0

    lax.fori_loop(0, PH, _phase, 0)


@jax.jit
def _run(offsets_ext, indices_pad, staged):
    mesh = plsc.VectorSubcoreMesh(core_axis_name="c", subcore_axis_name="s")
    scratch = (
        pltpu.VMEM((OFF_PAD,), jnp.int32),
        [[pltpu.VMEM((128,), jnp.int32) for _ in range(NSUB)]
         for _ in range(2)],
        [[pltpu.VMEM((128,), jnp.int32) for _ in range(NSUB)]
         for _ in range(2)],
        [pltpu.VMEM((CH, 2 * DIM), jnp.float32)
         for _ in range(2)],
        pltpu.VMEM((BPP + 8, 2 * DIM), jnp.float32),
        pltpu.VMEM((BPP + 16,), jnp.float32),
        pltpu.VMEM_SHARED((NS * BPP + 8, 2 * DIM), jnp.float32),
        [pltpu.SemaphoreType.DMA for _ in range(2)],
        [pltpu.SemaphoreType.DMA for _ in range(2)],
    )
    return pl.kernel(
        _body,
        out_type=jax.ShapeDtypeStruct((BATCH, 2 * DIM), jnp.float32),
        mesh=mesh,
        scratch_types=scratch,
        compiler_params=_params,
    )(offsets_ext, indices_pad, staged)


def kernel(indices, offsets, weight):
    offsets_ext = jnp.concatenate(
        [offsets, jnp.full((OFF_PAD,), TOTAL, jnp.int32)])
    indices_pad = jnp.concatenate(
        [indices, jnp.zeros((IDX_PAD,), jnp.int32)])
    staged = jnp.concatenate(
        [weight, jnp.zeros((VOCAB, DIM), jnp.float32)], axis=1)
    return _run(offsets_ext, indices_pad, staged)[:, :DIM]
